# Initial kernel scaffold; baseline (speedup 1.0000x reference)
import jax
import jax.numpy as jnp
from jax.experimental import pallas as pl


def kernel(points, walks, neg_samples, table):
    raise NotImplementedError("write your pallas kernel here")


# speedup vs baseline: 4.4171x; 4.4171x over previous
import functools

import jax
import jax.numpy as jnp
from jax import lax
from jax.experimental import pallas as pl
from jax.experimental.pallas import tpu as pltpu
from jax.experimental.pallas import tpu_sc as plsc

NUM_POINTS = 100000
EMBED = 128
B = 4096
WALK_LEN = 50
NUM_NEG = 20
NEG_PAD = 32
LANES = 16
DC = EMBED // LANES


def _rsqrt16(x):
    i = plsc.bitcast(x, jnp.int32)
    i = jnp.int32(0x5F3759DF) - (i >> 1)
    y = plsc.bitcast(i, jnp.float32)
    for _ in range(3):
        y = y * (1.5 - ((0.5 * x) * y) * y)
    return jnp.minimum(jnp.float32(1.0), y)


def _row_chunks(ref, r):
    return [ref[r, pl.ds(c * LANES, LANES)] for c in range(DC)]


def _norm_scale(chunks):
    nv = chunks[0] * chunks[0]
    for c in range(1, DC):
        nv = nv + chunks[c] * chunks[c]
    nsq = jnp.sum(nv)
    return _rsqrt16(jnp.full((LANES,), nsq, jnp.float32))


def _sc_body(points_hbm, walks_hbm, negs_hbm, table_hbm,
             pe_out, hs_out, sims_out,
             pidx, widx, nidx, prows, wrows, nrows, hs_scr, sims_scr,
             sem_p, sem_w, sem_n):
    info = plsc.get_sparse_core_info()
    nc = info.num_cores
    bpw = B // (nc * info.num_subcores)
    wid = lax.axis_index("s") * nc + lax.axis_index("c")
    base = pl.multiple_of(wid * bpw, bpw)

    pltpu.sync_copy(points_hbm.at[pl.ds(base, bpw)], pidx)
    pltpu.sync_copy(walks_hbm.at[pl.ds(base, bpw)], widx)
    pltpu.sync_copy(negs_hbm.at[pl.ds(base, bpw)], nidx)
    pltpu.async_copy(table_hbm.at[pidx], prows, sem_p).wait()

    def clip_point(b, _):
        ch = _row_chunks(prows, b)
        s = _norm_scale(ch)
        for c in range(DC):
            prows[b, pl.ds(c * LANES, LANES)] = ch[c] * s
        return 0

    lax.fori_loop(0, bpw, clip_point, 0)
    pltpu.sync_copy(prows, pe_out.at[pl.ds(base, bpw)])

    lane = lax.broadcasted_iota(jnp.int32, (LANES,), 0)

    def per_b(b, _):
        cw = pltpu.async_copy(table_hbm.at[widx.at[b]], wrows, sem_w)
        cn = pltpu.async_copy(table_hbm.at[nidx.at[b]], nrows, sem_n)
        cw.wait()
        cn.wait()
        ph = _row_chunks(prows, b)

        def walk(l, acc):
            wc = _row_chunks(wrows, l)
            s = _norm_scale(wc)
            return tuple(acc[c] + s * wc[c] for c in range(DC))

        acc = lax.fori_loop(
            0, WALK_LEN, walk, tuple(jnp.zeros((LANES,), jnp.float32) for _ in range(DC))
        )
        dv = ph[0] * acc[0]
        for c in range(1, DC):
            dv = dv + ph[c] * acc[c]
        hsum = jnp.full((LANES,), jnp.sum(dv), jnp.float32)
        hs_scr[b // LANES] = jnp.where(lane == (b % LANES), hsum, hs_scr[b // LANES])

        def neg(n, carry):
            v0, v1 = carry
            nch = _row_chunks(nrows, n)
            s = _norm_scale(nch)
            dvn = ph[0] * nch[0]
            for c in range(1, DC):
                dvn = dvn + ph[c] * nch[c]
            simv = jnp.full((LANES,), jnp.sum(dvn * s), jnp.float32)
            v0 = jnp.where(lane == n, simv, v0)
            v1 = jnp.where(lane == (n - LANES), simv, v1)
            return (v0, v1)

        zero = jnp.zeros((LANES,), jnp.float32)
        v0, v1 = lax.fori_loop(0, NUM_NEG, neg, (zero, zero))
        sims_scr[b, pl.ds(0, LANES)] = v0
        sims_scr[b, pl.ds(LANES, LANES)] = v1
        return 0

    lax.fori_loop(0, bpw, per_b, 0)
    pltpu.sync_copy(
        hs_scr, hs_out.at[pl.ds(pl.multiple_of(base // LANES, bpw // LANES), bpw // LANES)]
    )
    pltpu.sync_copy(sims_scr, sims_out.at[pl.ds(base, bpw)])


def _build_sc_kernel(bpw):
    mesh = plsc.VectorSubcoreMesh(core_axis_name="c", subcore_axis_name="s")
    return pl.kernel(
        _sc_body,
        mesh=mesh,
        compiler_params=pltpu.CompilerParams(needs_layout_passes=False),
        out_type=[
            jax.ShapeDtypeStruct((B, EMBED), jnp.float32),
            jax.ShapeDtypeStruct((B // LANES, LANES), jnp.float32),
            jax.ShapeDtypeStruct((B, NEG_PAD), jnp.float32),
        ],
        scratch_types=[
            pltpu.VMEM((bpw,), jnp.int32),
            pltpu.VMEM((bpw, WALK_LEN), jnp.int32),
            pltpu.VMEM((bpw, NUM_NEG), jnp.int32),
            pltpu.VMEM((bpw, EMBED), jnp.float32),
            pltpu.VMEM((WALK_LEN, EMBED), jnp.float32),
            pltpu.VMEM((NUM_NEG, EMBED), jnp.float32),
            pltpu.VMEM((bpw // LANES, LANES), jnp.float32),
            pltpu.VMEM((bpw, NEG_PAD), jnp.float32),
            pltpu.SemaphoreType.DMA,
            pltpu.SemaphoreType.DMA,
            pltpu.SemaphoreType.DMA,
        ],
    )


def _loss_body(sims_ref, hs_ref, out_ref):
    sims = sims_ref[...]
    mask = lax.broadcasted_iota(jnp.int32, (B, NEG_PAD), 1) < NUM_NEG
    e = jnp.where(mask, jnp.exp(jnp.where(mask, sims, 0.0)), 0.0)
    negsum = jnp.sum(e, axis=1, keepdims=True)
    out_ref[...] = jnp.sum(jnp.log(negsum) - hs_ref[...]).reshape(1, 1)


def kernel(points, walks, neg_samples, table):
    points = points.astype(jnp.int32)
    walks = walks.astype(jnp.int32)
    neg_samples = neg_samples.astype(jnp.int32)
    table = table.astype(jnp.float32)

    info = plsc.get_sparse_core_info()
    bpw = B // (info.num_cores * info.num_subcores)
    pe, hs, sims = _build_sc_kernel(bpw)(points, walks, neg_samples, table)

    loss = pl.pallas_call(
        _loss_body,
        out_shape=jax.ShapeDtypeStruct((1, 1), jnp.float32),
    )(sims, hs.reshape(B, 1))
    return loss[0, 0], pe


# speedup vs baseline: 7.4299x; 1.6821x over previous
import functools

import jax
import jax.numpy as jnp
from jax import lax
from jax.experimental import pallas as pl
from jax.experimental.pallas import tpu as pltpu
from jax.experimental.pallas import tpu_sc as plsc

NUM_POINTS = 100000
EMBED = 128
B = 4096
WALK_LEN = 50
NUM_NEG = 20
NEG_PAD = 32
LANES = 16
DC = EMBED // LANES


def _rsqrt16(x):
    i = plsc.bitcast(x, jnp.int32)
    i = jnp.int32(0x5F3759DF) - (i >> 1)
    y = plsc.bitcast(i, jnp.float32)
    for _ in range(3):
        y = y * (1.5 - ((0.5 * x) * y) * y)
    return jnp.minimum(jnp.float32(1.0), y)


def _row_chunks(ref, r):
    return [ref[r, pl.ds(c * LANES, LANES)] for c in range(DC)]


def _norm_scale(chunks):
    nv = chunks[0] * chunks[0]
    for c in range(1, DC):
        nv = nv + chunks[c] * chunks[c]
    nsq = jnp.sum(nv)
    return _rsqrt16(jnp.full((LANES,), nsq, jnp.float32))


WALK_ILV = 5
NEG_ILV = 4
PT_ILV = 4


def _sc_body(points_hbm, walks_hbm, negs_hbm, table_hbm,
             pe_out, hs_out, sims_out,
             pidx, widx, nidx, prows, wrows_a, nrows_a, wrows_b, nrows_b,
             hs_scr, sims_scr,
             sem_p, sem_wa, sem_na, sem_wb, sem_nb):
    info = plsc.get_sparse_core_info()
    nc = info.num_cores
    bpw = B // (nc * info.num_subcores)
    wid = lax.axis_index("s") * nc + lax.axis_index("c")
    base = pl.multiple_of(wid * bpw, bpw)

    pltpu.sync_copy(points_hbm.at[pl.ds(base, bpw)], pidx)
    pltpu.sync_copy(walks_hbm.at[pl.ds(base, bpw)], widx)
    pltpu.sync_copy(negs_hbm.at[pl.ds(base, bpw)], nidx)
    pltpu.async_copy(table_hbm.at[pidx], prows, sem_p).wait()

    def clip_point(i, _):
        for j in range(PT_ILV):
            b = i * PT_ILV + j
            ch = _row_chunks(prows, b)
            s = _norm_scale(ch)
            for c in range(DC):
                prows[b, pl.ds(c * LANES, LANES)] = ch[c] * s
        return 0

    lax.fori_loop(0, bpw // PT_ILV, clip_point, 0)
    pltpu.sync_copy(prows, pe_out.at[pl.ds(base, bpw)])

    lane = lax.broadcasted_iota(jnp.int32, (LANES,), 0)
    dummy_w = table_hbm.at[widx.at[0]]
    dummy_n = table_hbm.at[nidx.at[0]]

    def issue(b, wbuf, nbuf, sw, sn):
        pltpu.async_copy(table_hbm.at[widx.at[b]], wbuf, sw)
        pltpu.async_copy(table_hbm.at[nidx.at[b]], nbuf, sn)

    def drain(wbuf, nbuf, sw, sn):
        pltpu.make_async_copy(dummy_w, wbuf, sw).wait()
        pltpu.make_async_copy(dummy_n, nbuf, sn).wait()

    def compute(b, wrows, nrows):
        ph = _row_chunks(prows, b)

        def walk(i, acc):
            for j in range(WALK_ILV):
                l = i * WALK_ILV + j
                wc = _row_chunks(wrows, l)
                s = _norm_scale(wc)
                acc = tuple(acc[c] + s * wc[c] for c in range(DC))
            return acc

        acc = lax.fori_loop(
            0, WALK_LEN // WALK_ILV, walk,
            tuple(jnp.zeros((LANES,), jnp.float32) for _ in range(DC)),
        )
        dv = ph[0] * acc[0]
        for c in range(1, DC):
            dv = dv + ph[c] * acc[c]
        hsum = jnp.full((LANES,), jnp.sum(dv), jnp.float32)
        hs_scr[b // LANES] = jnp.where(lane == (b % LANES), hsum, hs_scr[b // LANES])

        def neg(i, carry):
            v0, v1 = carry
            for j in range(NEG_ILV):
                n = i * NEG_ILV + j
                nch = _row_chunks(nrows, n)
                s = _norm_scale(nch)
                dvn = ph[0] * nch[0]
                for c in range(1, DC):
                    dvn = dvn + ph[c] * nch[c]
                simv = jnp.full((LANES,), jnp.sum(dvn), jnp.float32) * s
                v0 = jnp.where(lane == n, simv, v0)
                v1 = jnp.where(lane == (n - LANES), simv, v1)
            return (v0, v1)

        zero = jnp.zeros((LANES,), jnp.float32)
        v0, v1 = lax.fori_loop(0, NUM_NEG // NEG_ILV, neg, (zero, zero))
        sims_scr[b, pl.ds(0, LANES)] = v0
        sims_scr[b, pl.ds(LANES, LANES)] = v1

    issue(0, wrows_a, nrows_a, sem_wa, sem_na)

    def pair(b2, _):
        b = b2 * 2
        issue(b + 1, wrows_b, nrows_b, sem_wb, sem_nb)
        drain(wrows_a, nrows_a, sem_wa, sem_na)
        compute(b, wrows_a, nrows_a)

        @pl.when(b2 + 1 < bpw // 2)
        def _():
            issue(b + 2, wrows_a, nrows_a, sem_wa, sem_na)

        drain(wrows_b, nrows_b, sem_wb, sem_nb)
        compute(b + 1, wrows_b, nrows_b)
        return 0

    lax.fori_loop(0, bpw // 2, pair, 0)
    pltpu.sync_copy(
        hs_scr, hs_out.at[pl.ds(pl.multiple_of(base // LANES, bpw // LANES), bpw // LANES)]
    )
    pltpu.sync_copy(sims_scr, sims_out.at[pl.ds(base, bpw)])


def _build_sc_kernel(bpw):
    mesh = plsc.VectorSubcoreMesh(core_axis_name="c", subcore_axis_name="s")
    return pl.kernel(
        _sc_body,
        mesh=mesh,
        compiler_params=pltpu.CompilerParams(needs_layout_passes=False),
        out_type=[
            jax.ShapeDtypeStruct((B, EMBED), jnp.float32),
            jax.ShapeDtypeStruct((B // LANES, LANES), jnp.float32),
            jax.ShapeDtypeStruct((B, NEG_PAD), jnp.float32),
        ],
        scratch_types=[
            pltpu.VMEM((bpw,), jnp.int32),
            pltpu.VMEM((bpw, WALK_LEN), jnp.int32),
            pltpu.VMEM((bpw, NUM_NEG), jnp.int32),
            pltpu.VMEM((bpw, EMBED), jnp.float32),
            pltpu.VMEM((WALK_LEN, EMBED), jnp.float32),
            pltpu.VMEM((NUM_NEG, EMBED), jnp.float32),
            pltpu.VMEM((WALK_LEN, EMBED), jnp.float32),
            pltpu.VMEM((NUM_NEG, EMBED), jnp.float32),
            pltpu.VMEM((bpw // LANES, LANES), jnp.float32),
            pltpu.VMEM((bpw, NEG_PAD), jnp.float32),
            pltpu.SemaphoreType.DMA,
            pltpu.SemaphoreType.DMA,
            pltpu.SemaphoreType.DMA,
            pltpu.SemaphoreType.DMA,
            pltpu.SemaphoreType.DMA,
        ],
    )


def _loss_body(sims_ref, hs_ref, out_ref):
    sims = sims_ref[...]
    mask = lax.broadcasted_iota(jnp.int32, (B, NEG_PAD), 1) < NUM_NEG
    e = jnp.where(mask, jnp.exp(jnp.where(mask, sims, 0.0)), 0.0)
    negsum = jnp.sum(e, axis=1, keepdims=True)
    out_ref[...] = jnp.sum(jnp.log(negsum) - hs_ref[...]).reshape(1, 1)


def kernel(points, walks, neg_samples, table):
    points = points.astype(jnp.int32)
    walks = walks.astype(jnp.int32)
    neg_samples = neg_samples.astype(jnp.int32)
    table = table.astype(jnp.float32)

    info = plsc.get_sparse_core_info()
    bpw = B // (info.num_cores * info.num_subcores)
    pe, hs, sims = _build_sc_kernel(bpw)(points, walks, neg_samples, table)

    loss = pl.pallas_call(
        _loss_body,
        out_shape=jax.ShapeDtypeStruct((1, 1), jnp.float32),
    )(sims, hs.reshape(B, 1))
    return loss[0, 0], pe
